# baseline (speedup 1.0000x reference)
import functools

import jax
import jax.numpy as jnp
from jax import lax
from jax.experimental import pallas as pl
from jax.experimental.pallas import tpu as pltpu
from jax.experimental.pallas import tpu_sc as plsc

_ALPHA = 0.2
_F32 = jnp.float32


def _leaky(x):
    return jnp.where(x >= 0, x, _ALPHA * x)


def _pad2(w, rows, cols):
    out = jnp.zeros((rows, cols), w.dtype)
    return out.at[: w.shape[0], : w.shape[1]].set(w)


def _mask8(adj):
    n = adj.shape[0]
    br = 512 if n % 512 == 0 else n

    def body(a_ref, m_ref):
        m_ref[...] = (a_ref[...] != 0.0).astype(jnp.int8)

    return pl.pallas_call(
        body,
        grid=(n // br,),
        in_specs=[pl.BlockSpec((br, n), lambda r: (r, 0))],
        out_specs=pl.BlockSpec((br, n), lambda r: (r, 0)),
        out_shape=jax.ShapeDtypeStruct((n, n), jnp.int8),
    )(adj)


def _proj(xin, W, a):
    n, din = xin.shape
    _, _, dout = W.shape
    br = 512 if n % 512 == 0 else n
    a1 = a[:, :dout, 0]
    a2 = a[:, dout:, 0]

    def body(x_ref, w0, w1, a10, a11, a20, a21, h0, h1, s0, s1, t0, t1):
        xv = x_ref[...]
        for w_ref, a1_ref, a2_ref, h_ref, s_ref, t_ref in (
            (w0, a10, a20, h0, s0, t0),
            (w1, a11, a21, h1, s1, t1),
        ):
            hk = jnp.dot(xv, w_ref[...], preferred_element_type=_F32)
            h_ref[...] = hk.astype(jnp.bfloat16)
            s_ref[0, :] = jnp.sum(hk * a1_ref[...], axis=1)
            t_ref[0, :] = jnp.sum(hk * a2_ref[...], axis=1)

    wspec = pl.BlockSpec((din, dout), lambda r: (0, 0))
    aspec = pl.BlockSpec((1, dout), lambda r: (0, 0))
    hspec = pl.BlockSpec((br, dout), lambda r: (r, 0))
    vspec = pl.BlockSpec((1, br), lambda r: (0, r))
    return pl.pallas_call(
        body,
        grid=(n // br,),
        in_specs=[pl.BlockSpec((br, din), lambda r: (r, 0)),
                  wspec, wspec, aspec, aspec, aspec, aspec],
        out_specs=[hspec, hspec, vspec, vspec, vspec, vspec],
        out_shape=[jax.ShapeDtypeStruct((n, dout), jnp.bfloat16)] * 2
        + [jax.ShapeDtypeStruct((1, n), _F32)] * 4,
    )(xin.astype(jnp.bfloat16), W.astype(jnp.bfloat16)[0],
      W.astype(jnp.bfloat16)[1], a1[0:1], a1[1:2], a2[0:1], a2[1:2])


def _attn(adj, h0, h1, s0, s1, t0, t1, bcat, dout):
    n = adj.shape[0]
    br = 1024 if n % 1024 == 0 else n
    bc = n
    grid = (n // br, n // bc)
    twod = 2 * dout

    def body(adj_ref, h0_ref, h1_ref, s0_ref, s1_ref, t0_ref, t1_ref, b_ref,
             out_ref, acc0, acc1, den0, den1):
        r = pl.program_id(0)
        c = pl.program_id(1)
        ncols = pl.num_programs(1)

        @pl.when(c == 0)
        def _():
            acc0[...] = jnp.zeros_like(acc0)
            acc1[...] = jnp.zeros_like(acc1)
            den0[...] = jnp.zeros_like(den0)
            den1[...] = jnp.zeros_like(den1)

        mask = adj_ref[...] != 0
        for s_ref, t_ref, h_ref, acc, den in (
            (s0_ref, t0_ref, h0_ref, acc0, den0),
            (s1_ref, t1_ref, h1_ref, acc1, den1),
        ):
            srow = s_ref[0, pl.ds(r * br, br)]
            tcol = t_ref[0, pl.ds(c * bc, bc)]
            tmax = jnp.max(t_ref[0, :])
            m = jnp.maximum(_leaky(srow + tmax), 0.0)
            bf = jnp.bfloat16
            e1 = jnp.exp(srow + tmax - m).astype(bf)
            e2 = jnp.exp(_ALPHA * (srow + tmax) - m).astype(bf)
            em = jnp.exp(-m).astype(bf)
            f1 = jnp.exp(tcol - tmax).astype(bf)
            f2 = jnp.exp(_ALPHA * (tcol - tmax)).astype(bf)
            p = jnp.maximum(e1[:, None] * f1[None, :],
                            e2[:, None] * f2[None, :])
            p = jnp.where(mask, p, em[:, None])
            acc[...] += jnp.dot(p, h_ref[...], preferred_element_type=_F32)
            den[...] += jnp.sum(p, axis=1, keepdims=True, dtype=_F32)

        @pl.when(c == ncols - 1)
        def _():
            for k, (acc, den) in enumerate(((acc0, den0), (acc1, den1))):
                o = _leaky(acc[...] / den[:, :1])
                nrm = jnp.maximum(
                    jnp.sqrt(jnp.sum(o * o, axis=1, keepdims=True)), 1e-12)
                out_ref[:, k * dout:(k + 1) * dout] = (
                    o / nrm + b_ref[0, k * dout:(k + 1) * dout][None, :])

    hspec = pl.BlockSpec((bc, dout), lambda r, c: (c, 0))
    vspec = pl.BlockSpec((1, n), lambda r, c: (0, 0))
    return pl.pallas_call(
        body,
        grid=grid,
        in_specs=[pl.BlockSpec((br, bc), lambda r, c: (r, c)),
                  hspec, hspec, vspec, vspec, vspec, vspec,
                  pl.BlockSpec((1, twod), lambda r, c: (0, 0))],
        out_specs=pl.BlockSpec((br, twod), lambda r, c: (r, 0)),
        out_shape=jax.ShapeDtypeStruct((n, twod), _F32),
        scratch_shapes=[
            pltpu.VMEM((br, dout), _F32),
            pltpu.VMEM((br, dout), _F32),
            pltpu.VMEM((br, 128), _F32),
            pltpu.VMEM((br, 128), _F32),
        ],
        compiler_params=pltpu.CompilerParams(
            dimension_semantics=("arbitrary", "arbitrary")),
    )(adj, h0, h1, s0, s1, t0, t1, bcat)


def _matmul_bias(xin, w, b):
    n, din = xin.shape
    dout = w.shape[1]
    br = 512 if n % 512 == 0 else n

    def body(x_ref, w_ref, b_ref, o_ref):
        o_ref[...] = (jnp.dot(x_ref[...], w_ref[...],
                              preferred_element_type=_F32) + b_ref[...])

    return pl.pallas_call(
        body,
        grid=(n // br,),
        in_specs=[pl.BlockSpec((br, din), lambda r: (r, 0)),
                  pl.BlockSpec((din, dout), lambda r: (0, 0)),
                  pl.BlockSpec((1, dout), lambda r: (0, 0))],
        out_specs=pl.BlockSpec((br, dout), lambda r: (r, 0)),
        out_shape=jax.ShapeDtypeStruct((n, dout), _F32),
    )(xin, w, b.reshape(1, -1))


def _tail(h4, tda_pad, gw, gb, gxw, gxb, gaw, gab, f1w, f1b, f2w, f2b,
          tf_ws, tg_ws):
    n, dh = h4.shape
    dct = dh + 128
    h3dim = tf_ws[2].shape[1]
    outdim = tf_ws[6].shape[1]
    br = 512 if n % 512 == 0 else n

    def body(h4_ref, td_ref, gw_ref, gb_ref, gxw_ref, gxb_ref, gaw_ref,
             gab_ref, f1w_ref, f1b_ref, f2w_ref, f2b_ref,
             aw1, ab1, aw2, ab2, aw3, ab3, awf, abf, ag1, an1, ag2, an2,
             ag3, an3,
             bw1, bb1, bw2, bb2, bw3, bb3, bwf, bbf, bg1, bn1, bg2, bn2,
             bg3, bn3,
             emb_ref):
        ct = jnp.concatenate([h4_ref[...], td_ref[...]], axis=1)
        g = jax.nn.sigmoid(
            jnp.dot(ct, gw_ref[...], preferred_element_type=_F32)
            + gb_ref[...])
        v = jnp.dot(ct, gxw_ref[...], preferred_element_type=_F32) + gxb_ref[...]
        fused = (jnp.dot(g * v, gaw_ref[...], preferred_element_type=_F32)
                 + gab_ref[...])
        mid = _leaky(jnp.dot(fused, f1w_ref[...], preferred_element_type=_F32)
                     + f1b_ref[...])
        fused = fused + jnp.dot(mid, f2w_ref[...],
                                preferred_element_type=_F32) + f2b_ref[...]

        def ln(xx, g_ref, b_ref):
            mu = jnp.mean(xx, axis=1, keepdims=True)
            xc = xx - mu
            var = jnp.mean(xc * xc, axis=1, keepdims=True)
            return xc / jnp.sqrt(var + 1e-5) * g_ref[...] + b_ref[...]

        def branch(w1, b1, w2, b2, w3, b3, wf, bf, g1, n1, g2, n2, g3, n3,
                   off):
            t = _leaky(ln(jnp.dot(fused, w1[...],
                                  preferred_element_type=_F32) + b1[...],
                          g1, n1))
            t = _leaky(ln(jnp.dot(t, w2[...],
                                  preferred_element_type=_F32) + b2[...],
                          g2, n2))
            t = _leaky(ln(jnp.dot(t, w3[...],
                                  preferred_element_type=_F32) + b3[...],
                          g3, n3))
            emb_ref[:, off:off + outdim] = jnp.dot(
                t, wf[...], preferred_element_type=_F32) + bf[...]

        branch(aw1, ab1, aw2, ab2, aw3, ab3, awf, abf, ag1, an1, ag2, an2,
               ag3, an3, 0)
        branch(bw1, bb1, bw2, bb2, bw3, bb3, bwf, bbf, bg1, bn1, bg2, bn2,
               bg3, bn3, outdim)

    def cspec(arr):
        return pl.BlockSpec(arr.shape, lambda r: tuple(0 for _ in arr.shape))

    consts = [gw, gb, gxw, gxb, gaw, gab, f1w, f1b, f2w, f2b] + list(tf_ws) \
        + list(tg_ws)
    return pl.pallas_call(
        body,
        grid=(n // br,),
        in_specs=[pl.BlockSpec((br, dh), lambda r: (r, 0)),
                  pl.BlockSpec((br, 128), lambda r: (r, 0))]
        + [cspec(a) for a in consts],
        out_specs=pl.BlockSpec((br, 2 * outdim), lambda r: (r, 0)),
        out_shape=jax.ShapeDtypeStruct((n, 2 * outdim), _F32),
    )(h4, tda_pad, *consts)


def _decode_sc(emb, idx0, idx1):
    nb = idx0.shape[0]
    d2 = emb.shape[1]
    d = d2 // 2
    nchunk = d // 16
    info = plsc.get_sparse_core_info()
    nw = info.num_cores * info.num_subcores
    bpw = nb // nw
    rr = 128
    rounds = bpw // rr
    mesh = plsc.VectorSubcoreMesh(core_axis_name="c", subcore_axis_name="s")

    @functools.partial(
        pl.kernel,
        mesh=mesh,
        out_type=jax.ShapeDtypeStruct((nb,), _F32),
        scratch_types=[
            pltpu.VMEM((rr,), jnp.int32),
            pltpu.VMEM((rr,), jnp.int32),
            pltpu.VMEM((rr, d2), _F32),
            pltpu.VMEM((rr, d2), _F32),
            pltpu.VMEM((bpw,), _F32),
            pltpu.SemaphoreType.DMA,
        ],
    )
    def k(emb_hbm, i0_hbm, i1_hbm, out_hbm, i0_v, i1_v, r0_v, r1_v,
          o_v, sem):
        wid = lax.axis_index("s") * info.num_cores + lax.axis_index("c")
        base = wid * bpw
        lane = lax.broadcasted_iota(jnp.int32, (16,), 0)
        gdn = lax.GatherDimensionNumbers(
            offset_dims=(), collapsed_slice_dims=(0,), start_index_map=(0,))

        def hsum16(v):
            for sh in (1, 2, 4, 8):
                perm = jnp.bitwise_xor(lane, sh)
                v = v + lax.gather(
                    v, perm[:, None], gdn, slice_sizes=(1,),
                    mode=lax.GatherScatterMode.PROMISE_IN_BOUNDS)
            return v

        for rnd in range(rounds):
            b0 = base + rnd * rr
            pltpu.sync_copy(i0_hbm.at[pl.ds(b0, rr)], i0_v)
            pltpu.sync_copy(i1_hbm.at[pl.ds(b0, rr)], i1_v)
            cp0 = pltpu.async_copy(emb_hbm.at[i0_v], r0_v, sem)
            cp1 = pltpu.async_copy(emb_hbm.at[i1_v], r1_v, sem)
            cp0.wait()
            cp1.wait()

            def group(gi, carry):
                vec = jnp.zeros((16,), _F32)
                for q in range(16):
                    p = gi * 16 + q
                    acc = r0_v[p, pl.ds(0, 16)] * r1_v[p, pl.ds(d, 16)]
                    for ch in range(1, nchunk):
                        acc = acc + (r0_v[p, pl.ds(ch * 16, 16)]
                                     * r1_v[p, pl.ds(d + ch * 16, 16)])
                    vec = jnp.where(lane == q, hsum16(acc), vec)
                o_v[pl.ds(rnd * rr + gi * 16, 16)] = vec
                return carry

            lax.fori_loop(0, rr // 16, group, 0)
        pltpu.sync_copy(o_v, out_hbm.at[pl.ds(base, bpw)])

    return k(emb, idx0, idx1)


def kernel(x, adj, train_sample, tda_feat, c1_W, c1_a, c1_b, c2_W, c2_a,
           c2_b, c3_W, c3_a, c3_b, c4_W, c4_a, c4_b, proj2_W, proj2_b,
           gate_W, gate_b, gatex_W, gatex_b, gatead_W, gatead_b, fp1_W,
           fp1_b, fp2_W, fp2_b, tf1_W, tf1_b, tf2_W, tf2_b, tf3_W, tf3_b,
           tff_W, tff_b, tf_ln1_g, tf_ln1_b, tf_ln2_g, tf_ln2_b, tf_ln3_g,
           tf_ln3_b, tg1_W, tg1_b, tg2_W, tg2_b, tg3_W, tg3_b, tgf_W,
           tgf_b, tg_ln1_g, tg_ln1_b, tg_ln2_g, tg_ln2_b, tg_ln3_g,
           tg_ln3_b):
    n = x.shape[0]

    mask8 = _mask8(adj)

    def gat_block(xin, W, a, b):
        dout = W.shape[2]
        h0, h1, s0, s1, t0, t1 = _proj(xin, W, a)
        return _attn(mask8, h0, h1, s0, s1, t0, t1, b.reshape(1, -1), dout)

    h = jax.nn.elu(gat_block(x, c1_W, c1_a, c1_b)) + x
    r2 = _matmul_bias(h, proj2_W, proj2_b)
    h2 = jax.nn.elu(gat_block(h, c2_W, c2_a, c2_b)) + r2
    h3 = jax.nn.elu(gat_block(h2, c3_W, c3_a, c3_b)) + h2
    h4 = jax.nn.elu(gat_block(h3, c4_W, c4_a, c4_b)) + h3

    dh = h4.shape[1]
    dct = dh + 128
    tda_pad = _pad2(tda_feat, n, 128)
    gw = _pad2(gate_W, dct, dct)
    gb = _pad2(gate_b.reshape(1, -1), 1, dct)
    gxw = _pad2(gatex_W, dct, dct)
    gxb = _pad2(gatex_b.reshape(1, -1), 1, dct)
    gaw = _pad2(gatead_W, dct, gatead_W.shape[1])
    tf_ws = (tf1_W, tf1_b.reshape(1, -1), tf2_W, tf2_b.reshape(1, -1),
             tf3_W, tf3_b.reshape(1, -1), tff_W, tff_b.reshape(1, -1),
             tf_ln1_g.reshape(1, -1), tf_ln1_b.reshape(1, -1),
             tf_ln2_g.reshape(1, -1), tf_ln2_b.reshape(1, -1),
             tf_ln3_g.reshape(1, -1), tf_ln3_b.reshape(1, -1))
    tg_ws = (tg1_W, tg1_b.reshape(1, -1), tg2_W, tg2_b.reshape(1, -1),
             tg3_W, tg3_b.reshape(1, -1), tgf_W, tgf_b.reshape(1, -1),
             tg_ln1_g.reshape(1, -1), tg_ln1_b.reshape(1, -1),
             tg_ln2_g.reshape(1, -1), tg_ln2_b.reshape(1, -1),
             tg_ln3_g.reshape(1, -1), tg_ln3_b.reshape(1, -1))
    emb = _tail(h4, tda_pad, gw, gb, gxw, gxb, gaw,
                gatead_b.reshape(1, -1), fp1_W, fp1_b.reshape(1, -1),
                fp2_W, fp2_b.reshape(1, -1), tf_ws, tg_ws)

    idx0 = train_sample[:, 0]
    idx1 = train_sample[:, 1]
    return _decode_sc(emb, idx0, idx1)


# speedup vs baseline: 1.1172x; 1.1172x over previous
import functools

import jax
import jax.numpy as jnp
from jax import lax
from jax.experimental import pallas as pl
from jax.experimental.pallas import tpu as pltpu
from jax.experimental.pallas import tpu_sc as plsc

_ALPHA = 0.2
_F32 = jnp.float32


def _leaky(x):
    return jnp.where(x >= 0, x, _ALPHA * x)


def _pad2(w, rows, cols):
    out = jnp.zeros((rows, cols), w.dtype)
    return out.at[: w.shape[0], : w.shape[1]].set(w)


def _mask8(adj):
    n = adj.shape[0]
    br = 512 if n % 512 == 0 else n

    def body(a_ref, m_ref):
        m_ref[...] = (a_ref[...] != 0.0).astype(jnp.int8)

    return pl.pallas_call(
        body,
        grid=(n // br,),
        in_specs=[pl.BlockSpec((br, n), lambda r: (r, 0))],
        out_specs=pl.BlockSpec((br, n), lambda r: (r, 0)),
        out_shape=jax.ShapeDtypeStruct((n, n), jnp.int8),
    )(adj)


def _proj(att, res, W, a, p2w=None, p2b=None):
    n, din = res.shape
    _, _, dout = W.shape
    br = 512 if n % 512 == 0 else n
    bf = jnp.bfloat16
    a1 = a[:, :dout, 0]
    a2 = a[:, dout:, 0]
    has_att = att is not None
    has_p2 = p2w is not None
    emit_comb = has_att and not has_p2

    def body(*refs):
        i = 0
        if has_att:
            att_ref = refs[i]
            i += 1
        res_ref = refs[i]
        w0, w1, a10, a11, a20, a21 = refs[i + 1:i + 7]
        i += 7
        if has_p2:
            p2w_ref, p2b_ref = refs[i:i + 2]
            i += 2
        if emit_comb:
            comb_ref = refs[i]
            i += 1
        h0, h1, s0, s1, t0, t1 = refs[i:i + 6]
        i += 6
        if has_p2:
            r2_ref = refs[i]

        if has_att:
            av = att_ref[...]
            xv = jnp.where(av > 0, av, jnp.exp(av) - 1.0) + res_ref[...]
            if emit_comb:
                comb_ref[...] = xv
        else:
            xv = res_ref[...]
        xb = xv.astype(bf)
        for w_ref, a1_ref, a2_ref, h_ref, s_ref, t_ref in (
            (w0, a10, a20, h0, s0, t0),
            (w1, a11, a21, h1, s1, t1),
        ):
            hk = jnp.dot(xb, w_ref[...], preferred_element_type=_F32)
            h_ref[...] = hk.astype(bf)
            s_ref[0, :] = jnp.sum(hk * a1_ref[...], axis=1)
            t_ref[0, :] = jnp.sum(hk * a2_ref[...], axis=1)
        if has_p2:
            r2_ref[...] = (jnp.dot(xv, p2w_ref[...],
                                   preferred_element_type=_F32)
                           + p2b_ref[...])

    xspec = pl.BlockSpec((br, din), lambda r: (r, 0))
    wspec = pl.BlockSpec((din, dout), lambda r: (0, 0))
    aspec = pl.BlockSpec((1, dout), lambda r: (0, 0))
    hspec = pl.BlockSpec((br, dout), lambda r: (r, 0))
    vspec = pl.BlockSpec((1, br), lambda r: (0, r))

    in_specs = ([xspec] if has_att else []) + [
        xspec, wspec, wspec, aspec, aspec, aspec, aspec]
    args = ([att] if has_att else []) + [
        res, W.astype(bf)[0], W.astype(bf)[1],
        a1[0:1], a1[1:2], a2[0:1], a2[1:2]]
    out_specs = [hspec, hspec, vspec, vspec, vspec, vspec]
    out_shape = [jax.ShapeDtypeStruct((n, dout), bf)] * 2 \
        + [jax.ShapeDtypeStruct((1, n), _F32)] * 4
    if emit_comb:
        out_specs = [xspec] + out_specs
        out_shape = [jax.ShapeDtypeStruct((n, din), _F32)] + out_shape
    if has_p2:
        dp = p2w.shape[1]
        in_specs += [pl.BlockSpec((din, dp), lambda r: (0, 0)),
                     pl.BlockSpec((1, dp), lambda r: (0, 0))]
        args += [p2w, p2b.reshape(1, -1)]
        out_specs = out_specs + [pl.BlockSpec((br, dp), lambda r: (r, 0))]
        out_shape = out_shape + [jax.ShapeDtypeStruct((n, dp), _F32)]

    return pl.pallas_call(
        body,
        grid=(n // br,),
        in_specs=in_specs,
        out_specs=out_specs,
        out_shape=out_shape,
    )(*args)


def _attn(adj, h0, h1, s0, s1, t0, t1, bcat, dout):
    n = adj.shape[0]
    br = 1024 if n % 1024 == 0 else n
    bc = 2048 if n % 2048 == 0 else br
    grid = (n // br, n // bc)
    twod = 2 * dout

    def body(adj_ref, h0_ref, h1_ref, s0_ref, s1_ref, t0_ref, t1_ref, b_ref,
             out_ref, acc0, acc1, den0, den1):
        r = pl.program_id(0)
        c = pl.program_id(1)
        ncols = pl.num_programs(1)

        @pl.when(c == 0)
        def _():
            acc0[...] = jnp.zeros_like(acc0)
            acc1[...] = jnp.zeros_like(acc1)
            den0[...] = jnp.zeros_like(den0)
            den1[...] = jnp.zeros_like(den1)

        mask = adj_ref[...] != 0
        for s_ref, t_ref, h_ref, acc, den in (
            (s0_ref, t0_ref, h0_ref, acc0, den0),
            (s1_ref, t1_ref, h1_ref, acc1, den1),
        ):
            srow = s_ref[0, pl.ds(r * br, br)]
            tcol = t_ref[0, pl.ds(c * bc, bc)]
            tmax = jnp.max(t_ref[0, :])
            m = jnp.maximum(_leaky(srow + tmax), 0.0)
            bf = jnp.bfloat16
            e1 = jnp.exp(srow + tmax - m).astype(bf)
            e2 = jnp.exp(_ALPHA * (srow + tmax) - m).astype(bf)
            em = jnp.exp(-m).astype(bf)
            f1 = jnp.exp(tcol - tmax).astype(bf)
            f2 = jnp.exp(_ALPHA * (tcol - tmax)).astype(bf)
            p = jnp.maximum(e1[:, None] * f1[None, :],
                            e2[:, None] * f2[None, :])
            p = jnp.where(mask, p, em[:, None])
            acc[...] += jnp.dot(p, h_ref[...], preferred_element_type=_F32)
            den[...] += jnp.sum(p, axis=1, keepdims=True, dtype=_F32)

        @pl.when(c == ncols - 1)
        def _():
            for k, (acc, den) in enumerate(((acc0, den0), (acc1, den1))):
                o = _leaky(acc[...] / den[:, :1])
                nrm = jnp.maximum(
                    jnp.sqrt(jnp.sum(o * o, axis=1, keepdims=True)), 1e-12)
                out_ref[:, k * dout:(k + 1) * dout] = (
                    o / nrm + b_ref[0, k * dout:(k + 1) * dout][None, :])

    hspec = pl.BlockSpec((bc, dout), lambda r, c: (c, 0))
    vspec = pl.BlockSpec((1, n), lambda r, c: (0, 0))
    return pl.pallas_call(
        body,
        grid=grid,
        in_specs=[pl.BlockSpec((br, bc), lambda r, c: (r, c)),
                  hspec, hspec, vspec, vspec, vspec, vspec,
                  pl.BlockSpec((1, twod), lambda r, c: (0, 0))],
        out_specs=pl.BlockSpec((br, twod), lambda r, c: (r, 0)),
        out_shape=jax.ShapeDtypeStruct((n, twod), _F32),
        scratch_shapes=[
            pltpu.VMEM((br, dout), _F32),
            pltpu.VMEM((br, dout), _F32),
            pltpu.VMEM((br, 128), _F32),
            pltpu.VMEM((br, 128), _F32),
        ],
        compiler_params=pltpu.CompilerParams(
            dimension_semantics=("arbitrary", "arbitrary")),
    )(adj, h0, h1, s0, s1, t0, t1, bcat)


def _tail(att4, h3, tda_pad, gw, gb, gxw, gxb, gaw, gab, f1w, f1b, f2w, f2b,
          tf_ws, tg_ws):
    n, dh = att4.shape
    outdim = tf_ws[6].shape[1]
    br = 512 if n % 512 == 0 else n
    bf = jnp.bfloat16

    def bdot(x, w_ref):
        return jnp.dot(x, w_ref[...], preferred_element_type=_F32)

    def body(a4_ref, h3_ref, td_ref, gw_ref, gb_ref, gxw_ref, gxb_ref,
             gaw_ref, gab_ref, f1w_ref, f1b_ref, f2w_ref, f2b_ref,
             aw1, ab1, aw2, ab2, aw3, ab3, awf, abf, ag1, an1, ag2, an2,
             ag3, an3,
             bw1, bb1, bw2, bb2, bw3, bb3, bwf, bbf, bg1, bn1, bg2, bn2,
             bg3, bn3,
             emb_ref):
        av = a4_ref[...]
        h4 = jnp.where(av > 0, av, jnp.exp(av) - 1.0) + h3_ref[...]
        ct = jnp.concatenate([h4, td_ref[...]], axis=1)
        g = jax.nn.sigmoid(bdot(ct, gw_ref) + gb_ref[...])
        v = bdot(ct, gxw_ref) + gxb_ref[...]
        fused = bdot(g * v, gaw_ref) + gab_ref[...]
        mid = _leaky(bdot(fused, f1w_ref) + f1b_ref[...])
        fused = fused + bdot(mid, f2w_ref) + f2b_ref[...]

        def ln(xx, g_ref, b_ref):
            mu = jnp.mean(xx, axis=1, keepdims=True)
            xc = xx - mu
            var = jnp.mean(xc * xc, axis=1, keepdims=True)
            return xc / jnp.sqrt(var + 1e-5) * g_ref[...] + b_ref[...]

        def branch(w1, b1, w2, b2, w3, b3, wf, bfi, g1, n1, g2, n2, g3, n3,
                   off):
            t = _leaky(ln(bdot(fused, w1) + b1[...], g1, n1))
            t = _leaky(ln(bdot(t, w2) + b2[...], g2, n2))
            t = _leaky(ln(bdot(t, w3) + b3[...], g3, n3))
            emb_ref[:, off:off + outdim] = bdot(t, wf) + bfi[...]

        branch(aw1, ab1, aw2, ab2, aw3, ab3, awf, abf, ag1, an1, ag2, an2,
               ag3, an3, 0)
        branch(bw1, bb1, bw2, bb2, bw3, bb3, bwf, bbf, bg1, bn1, bg2, bn2,
               bg3, bn3, outdim)

    def cspec(arr):
        return pl.BlockSpec(arr.shape, lambda r: tuple(0 for _ in arr.shape))

    consts = [gw, gb, gxw, gxb, gaw, gab, f1w, f1b, f2w, f2b] \
        + list(tf_ws) + list(tg_ws)
    return pl.pallas_call(
        body,
        grid=(n // br,),
        in_specs=[pl.BlockSpec((br, dh), lambda r: (r, 0)),
                  pl.BlockSpec((br, dh), lambda r: (r, 0)),
                  pl.BlockSpec((br, 128), lambda r: (r, 0))]
        + [cspec(a) for a in consts],
        out_specs=pl.BlockSpec((br, 2 * outdim), lambda r: (r, 0)),
        out_shape=jax.ShapeDtypeStruct((n, 2 * outdim), _F32),
    )(att4, h3, tda_pad, *consts)


def _decode_sc(emb, idx0, idx1):
    nb = idx0.shape[0]
    d2 = emb.shape[1]
    d = d2 // 2
    nchunk = d // 16
    info = plsc.get_sparse_core_info()
    nw = info.num_cores * info.num_subcores
    bpw = nb // nw
    rr = 128
    rounds = bpw // rr
    mesh = plsc.VectorSubcoreMesh(core_axis_name="c", subcore_axis_name="s")

    @functools.partial(
        pl.kernel,
        mesh=mesh,
        out_type=jax.ShapeDtypeStruct((nb,), _F32),
        scratch_types=[
            pltpu.VMEM((rr,), jnp.int32),
            pltpu.VMEM((rr,), jnp.int32),
            pltpu.VMEM((rr,), jnp.int32),
            pltpu.VMEM((rr,), jnp.int32),
            pltpu.VMEM((rr, d2), _F32),
            pltpu.VMEM((rr, d2), _F32),
            pltpu.VMEM((rr, d2), _F32),
            pltpu.VMEM((rr, d2), _F32),
            pltpu.VMEM((bpw,), _F32),
            pltpu.SemaphoreType.DMA,
            pltpu.SemaphoreType.DMA,
        ],
    )
    def k(emb_hbm, i0_hbm, i1_hbm, out_hbm, i0a, i1a, i0b, i1b,
          r0a, r1a, r0b, r1b, o_v, sema, semb):
        wid = lax.axis_index("s") * info.num_cores + lax.axis_index("c")
        base = wid * bpw
        lane = lax.broadcasted_iota(jnp.int32, (16,), 0)
        gdn = lax.GatherDimensionNumbers(
            offset_dims=(), collapsed_slice_dims=(0,), start_index_map=(0,))
        bufs = ((i0a, i1a, r0a, r1a, sema), (i0b, i1b, r0b, r1b, semb))

        def hsum16(v):
            for sh in (1, 2, 4, 8):
                perm = jnp.bitwise_xor(lane, sh)
                v = v + lax.gather(
                    v, perm[:, None], gdn, slice_sizes=(1,),
                    mode=lax.GatherScatterMode.PROMISE_IN_BOUNDS)
            return v

        def start(rnd, bset):
            i0_v, i1_v, r0_v, r1_v, sem = bset
            b0 = base + rnd * rr
            pltpu.sync_copy(i0_hbm.at[pl.ds(b0, rr)], i0_v)
            pltpu.sync_copy(i1_hbm.at[pl.ds(b0, rr)], i1_v)
            cp0 = pltpu.async_copy(emb_hbm.at[i0_v], r0_v, sem)
            cp1 = pltpu.async_copy(emb_hbm.at[i1_v], r1_v, sem)
            return cp0, cp1

        pending = start(0, bufs[0])
        for rnd in range(rounds):
            cp0, cp1 = pending
            cp0.wait()
            cp1.wait()
            if rnd + 1 < rounds:
                pending = start(rnd + 1, bufs[(rnd + 1) % 2])
            _, _, r0_v, r1_v, _ = bufs[rnd % 2]

            def group(gi, carry):
                vec = jnp.zeros((16,), _F32)
                for q in range(16):
                    p = gi * 16 + q
                    acc = r0_v[p, pl.ds(0, 16)] * r1_v[p, pl.ds(d, 16)]
                    for ch in range(1, nchunk):
                        acc = acc + (r0_v[p, pl.ds(ch * 16, 16)]
                                     * r1_v[p, pl.ds(d + ch * 16, 16)])
                    vec = jnp.where(lane == q, hsum16(acc), vec)
                o_v[pl.ds(rnd * rr + gi * 16, 16)] = vec
                return carry

            lax.fori_loop(0, rr // 16, group, 0)
        pltpu.sync_copy(o_v, out_hbm.at[pl.ds(base, bpw)])

    return k(emb, idx0, idx1)


def kernel(x, adj, train_sample, tda_feat, c1_W, c1_a, c1_b, c2_W, c2_a,
           c2_b, c3_W, c3_a, c3_b, c4_W, c4_a, c4_b, proj2_W, proj2_b,
           gate_W, gate_b, gatex_W, gatex_b, gatead_W, gatead_b, fp1_W,
           fp1_b, fp2_W, fp2_b, tf1_W, tf1_b, tf2_W, tf2_b, tf3_W, tf3_b,
           tff_W, tff_b, tf_ln1_g, tf_ln1_b, tf_ln2_g, tf_ln2_b, tf_ln3_g,
           tf_ln3_b, tg1_W, tg1_b, tg2_W, tg2_b, tg3_W, tg3_b, tgf_W,
           tgf_b, tg_ln1_g, tg_ln1_b, tg_ln2_g, tg_ln2_b, tg_ln3_g,
           tg_ln3_b):
    n = x.shape[0]

    mask8 = _mask8(adj)

    def attn(hs, b, dout):
        h0, h1, s0, s1, t0, t1 = hs
        return _attn(mask8, h0, h1, s0, s1, t0, t1, b.reshape(1, -1), dout)

    hs = _proj(None, x, c1_W, c1_a)
    att1 = attn(hs, c1_b, c1_W.shape[2])
    *hs, r2 = _proj(att1, x, c2_W, c2_a, proj2_W, proj2_b)
    att2 = attn(hs, c2_b, c2_W.shape[2])
    h2, *hs = _proj(att2, r2, c3_W, c3_a)
    att3 = attn(hs, c3_b, c3_W.shape[2])
    h3, *hs = _proj(att3, h2, c4_W, c4_a)
    att4 = attn(hs, c4_b, c4_W.shape[2])

    dh = att4.shape[1]
    dct = dh + 128
    tda_pad = _pad2(tda_feat, n, 128)
    gw = _pad2(gate_W, dct, dct)
    gb = _pad2(gate_b.reshape(1, -1), 1, dct)
    gxw = _pad2(gatex_W, dct, dct)
    gxb = _pad2(gatex_b.reshape(1, -1), 1, dct)
    gaw = _pad2(gatead_W, dct, gatead_W.shape[1])
    tf_ws = (tf1_W, tf1_b.reshape(1, -1), tf2_W, tf2_b.reshape(1, -1),
             tf3_W, tf3_b.reshape(1, -1), tff_W, tff_b.reshape(1, -1),
             tf_ln1_g.reshape(1, -1), tf_ln1_b.reshape(1, -1),
             tf_ln2_g.reshape(1, -1), tf_ln2_b.reshape(1, -1),
             tf_ln3_g.reshape(1, -1), tf_ln3_b.reshape(1, -1))
    tg_ws = (tg1_W, tg1_b.reshape(1, -1), tg2_W, tg2_b.reshape(1, -1),
             tg3_W, tg3_b.reshape(1, -1), tgf_W, tgf_b.reshape(1, -1),
             tg_ln1_g.reshape(1, -1), tg_ln1_b.reshape(1, -1),
             tg_ln2_g.reshape(1, -1), tg_ln2_b.reshape(1, -1),
             tg_ln3_g.reshape(1, -1), tg_ln3_b.reshape(1, -1))
    emb = _tail(att4, h3, tda_pad, gw, gb, gxw, gxb, gaw,
                gatead_b.reshape(1, -1), fp1_W, fp1_b.reshape(1, -1),
                fp2_W, fp2_b.reshape(1, -1), tf_ws, tg_ws)

    idx0 = train_sample[:, 0]
    idx1 = train_sample[:, 1]
    return _decode_sc(emb, idx0, idx1)


# speedup vs baseline: 1.1255x; 1.0074x over previous
import functools

import jax
import jax.numpy as jnp
from jax import lax
from jax.experimental import pallas as pl
from jax.experimental.pallas import tpu as pltpu
from jax.experimental.pallas import tpu_sc as plsc

_ALPHA = 0.2
_F32 = jnp.float32


def _leaky(x):
    return jnp.where(x >= 0, x, _ALPHA * x)


def _pad2(w, rows, cols):
    out = jnp.zeros((rows, cols), w.dtype)
    return out.at[: w.shape[0], : w.shape[1]].set(w)


def _mask8(adj):
    n = adj.shape[0]
    br = 512 if n % 512 == 0 else n

    def body(a_ref, m_ref):
        m_ref[...] = (a_ref[...] != 0.0).astype(jnp.int8)

    return pl.pallas_call(
        body,
        grid=(n // br,),
        in_specs=[pl.BlockSpec((br, n), lambda r: (r, 0))],
        out_specs=pl.BlockSpec((br, n), lambda r: (r, 0)),
        out_shape=jax.ShapeDtypeStruct((n, n), jnp.int8),
    )(adj)


def _proj(att, res, W, a, p2w=None, p2b=None):
    n, din = res.shape
    _, _, dout = W.shape
    br = 512 if n % 512 == 0 else n
    bf = jnp.bfloat16
    a1 = a[:, :dout, 0]
    a2 = a[:, dout:, 0]
    has_att = att is not None
    has_p2 = p2w is not None
    emit_comb = has_att and not has_p2

    def body(*refs):
        i = 0
        if has_att:
            att_ref = refs[i]
            i += 1
        res_ref = refs[i]
        w0, w1, a10, a11, a20, a21 = refs[i + 1:i + 7]
        i += 7
        if has_p2:
            p2w_ref, p2b_ref = refs[i:i + 2]
            i += 2
        if emit_comb:
            comb_ref = refs[i]
            i += 1
        h0, h1, s0, s1, t0, t1 = refs[i:i + 6]
        i += 6
        if has_p2:
            r2_ref = refs[i]

        if has_att:
            av = att_ref[...]
            xv = jnp.where(av > 0, av, jnp.exp(av) - 1.0) + res_ref[...]
            if emit_comb:
                comb_ref[...] = xv
        else:
            xv = res_ref[...]
        xb = xv.astype(bf)
        for w_ref, a1_ref, a2_ref, h_ref, s_ref, t_ref in (
            (w0, a10, a20, h0, s0, t0),
            (w1, a11, a21, h1, s1, t1),
        ):
            hk = jnp.dot(xb, w_ref[...], preferred_element_type=_F32)
            h_ref[...] = hk.astype(bf)
            s_ref[0, :] = jnp.sum(hk * a1_ref[...], axis=1)
            t_ref[0, :] = jnp.sum(hk * a2_ref[...], axis=1)
        if has_p2:
            r2_ref[...] = (jnp.dot(xv, p2w_ref[...],
                                   preferred_element_type=_F32)
                           + p2b_ref[...])

    xspec = pl.BlockSpec((br, din), lambda r: (r, 0))
    wspec = pl.BlockSpec((din, dout), lambda r: (0, 0))
    aspec = pl.BlockSpec((1, dout), lambda r: (0, 0))
    hspec = pl.BlockSpec((br, dout), lambda r: (r, 0))
    vspec = pl.BlockSpec((1, br), lambda r: (0, r))

    in_specs = ([xspec] if has_att else []) + [
        xspec, wspec, wspec, aspec, aspec, aspec, aspec]
    args = ([att] if has_att else []) + [
        res, W.astype(bf)[0], W.astype(bf)[1],
        a1[0:1], a1[1:2], a2[0:1], a2[1:2]]
    out_specs = [hspec, hspec, vspec, vspec, vspec, vspec]
    out_shape = [jax.ShapeDtypeStruct((n, dout), bf)] * 2 \
        + [jax.ShapeDtypeStruct((1, n), _F32)] * 4
    if emit_comb:
        out_specs = [xspec] + out_specs
        out_shape = [jax.ShapeDtypeStruct((n, din), _F32)] + out_shape
    if has_p2:
        dp = p2w.shape[1]
        in_specs += [pl.BlockSpec((din, dp), lambda r: (0, 0)),
                     pl.BlockSpec((1, dp), lambda r: (0, 0))]
        args += [p2w, p2b.reshape(1, -1)]
        out_specs = out_specs + [pl.BlockSpec((br, dp), lambda r: (r, 0))]
        out_shape = out_shape + [jax.ShapeDtypeStruct((n, dp), _F32)]

    return pl.pallas_call(
        body,
        grid=(n // br,),
        in_specs=in_specs,
        out_specs=out_specs,
        out_shape=out_shape,
    )(*args)


def _attn(adj, h0, h1, s0, s1, t0, t1, bcat, dout):
    n = adj.shape[0]
    br = 2048 if n % 2048 == 0 else n
    bc = 2048 if n % 2048 == 0 else br
    grid = (n // br, n // bc)
    twod = 2 * dout

    def body(adj_ref, h0_ref, h1_ref, s0_ref, s1_ref, t0_ref, t1_ref, b_ref,
             out_ref, acc0, acc1, den0, den1):
        r = pl.program_id(0)
        c = pl.program_id(1)
        ncols = pl.num_programs(1)

        @pl.when(c == 0)
        def _():
            acc0[...] = jnp.zeros_like(acc0)
            acc1[...] = jnp.zeros_like(acc1)
            den0[...] = jnp.zeros_like(den0)
            den1[...] = jnp.zeros_like(den1)

        mask = adj_ref[...] != 0
        for s_ref, t_ref, h_ref, acc, den in (
            (s0_ref, t0_ref, h0_ref, acc0, den0),
            (s1_ref, t1_ref, h1_ref, acc1, den1),
        ):
            srow = s_ref[0, pl.ds(r * br, br)]
            tcol = t_ref[0, pl.ds(c * bc, bc)]
            tmax = jnp.max(t_ref[0, :])
            m = jnp.maximum(_leaky(srow + tmax), 0.0)
            bf = jnp.bfloat16
            e1 = jnp.exp(srow + tmax - m).astype(bf)
            e2 = jnp.exp(_ALPHA * (srow + tmax) - m).astype(bf)
            em = jnp.exp(-m).astype(bf)
            f1 = jnp.exp(tcol - tmax).astype(bf)
            f2 = jnp.exp(_ALPHA * (tcol - tmax)).astype(bf)
            p = jnp.maximum(e1[:, None] * f1[None, :],
                            e2[:, None] * f2[None, :])
            p = jnp.where(mask, p, em[:, None])
            acc[...] += jnp.dot(p, h_ref[...], preferred_element_type=_F32)
            den[...] += jnp.sum(p, axis=1, keepdims=True, dtype=_F32)

        @pl.when(c == ncols - 1)
        def _():
            for k, (acc, den) in enumerate(((acc0, den0), (acc1, den1))):
                o = _leaky(acc[...] / den[:, :1])
                nrm = jnp.maximum(
                    jnp.sqrt(jnp.sum(o * o, axis=1, keepdims=True)), 1e-12)
                out_ref[:, k * dout:(k + 1) * dout] = (
                    o / nrm + b_ref[0, k * dout:(k + 1) * dout][None, :])

    hspec = pl.BlockSpec((bc, dout), lambda r, c: (c, 0))
    vspec = pl.BlockSpec((1, n), lambda r, c: (0, 0))
    return pl.pallas_call(
        body,
        grid=grid,
        in_specs=[pl.BlockSpec((br, bc), lambda r, c: (r, c)),
                  hspec, hspec, vspec, vspec, vspec, vspec,
                  pl.BlockSpec((1, twod), lambda r, c: (0, 0))],
        out_specs=pl.BlockSpec((br, twod), lambda r, c: (r, 0)),
        out_shape=jax.ShapeDtypeStruct((n, twod), _F32),
        scratch_shapes=[
            pltpu.VMEM((br, dout), _F32),
            pltpu.VMEM((br, dout), _F32),
            pltpu.VMEM((br, 128), _F32),
            pltpu.VMEM((br, 128), _F32),
        ],
        compiler_params=pltpu.CompilerParams(
            dimension_semantics=("arbitrary", "arbitrary")),
    )(adj, h0, h1, s0, s1, t0, t1, bcat)


def _tail(att4, h3, tda_pad, gw, gb, gxw, gxb, gaw, gab, f1w, f1b, f2w, f2b,
          tf_ws, tg_ws):
    n, dh = att4.shape
    outdim = tf_ws[6].shape[1]
    br = 512 if n % 512 == 0 else n
    bf = jnp.bfloat16

    def bdot(x, w_ref):
        return jnp.dot(x, w_ref[...], preferred_element_type=_F32)

    def body(a4_ref, h3_ref, td_ref, gw_ref, gb_ref, gxw_ref, gxb_ref,
             gaw_ref, gab_ref, f1w_ref, f1b_ref, f2w_ref, f2b_ref,
             aw1, ab1, aw2, ab2, aw3, ab3, awf, abf, ag1, an1, ag2, an2,
             ag3, an3,
             bw1, bb1, bw2, bb2, bw3, bb3, bwf, bbf, bg1, bn1, bg2, bn2,
             bg3, bn3,
             emb_ref):
        av = a4_ref[...]
        h4 = jnp.where(av > 0, av, jnp.exp(av) - 1.0) + h3_ref[...]
        ct = jnp.concatenate([h4, td_ref[...]], axis=1)
        g = jax.nn.sigmoid(bdot(ct, gw_ref) + gb_ref[...])
        v = bdot(ct, gxw_ref) + gxb_ref[...]
        fused = bdot(g * v, gaw_ref) + gab_ref[...]
        mid = _leaky(bdot(fused, f1w_ref) + f1b_ref[...])
        fused = fused + bdot(mid, f2w_ref) + f2b_ref[...]

        def ln(xx, g_ref, b_ref):
            mu = jnp.mean(xx, axis=1, keepdims=True)
            xc = xx - mu
            var = jnp.mean(xc * xc, axis=1, keepdims=True)
            return xc / jnp.sqrt(var + 1e-5) * g_ref[...] + b_ref[...]

        def branch(w1, b1, w2, b2, w3, b3, wf, bfi, g1, n1, g2, n2, g3, n3,
                   off):
            t = _leaky(ln(bdot(fused, w1) + b1[...], g1, n1))
            t = _leaky(ln(bdot(t, w2) + b2[...], g2, n2))
            t = _leaky(ln(bdot(t, w3) + b3[...], g3, n3))
            emb_ref[:, off:off + outdim] = bdot(t, wf) + bfi[...]

        branch(aw1, ab1, aw2, ab2, aw3, ab3, awf, abf, ag1, an1, ag2, an2,
               ag3, an3, 0)
        branch(bw1, bb1, bw2, bb2, bw3, bb3, bwf, bbf, bg1, bn1, bg2, bn2,
               bg3, bn3, outdim)

    def cspec(arr):
        return pl.BlockSpec(arr.shape, lambda r: tuple(0 for _ in arr.shape))

    consts = [gw, gb, gxw, gxb, gaw, gab, f1w, f1b, f2w, f2b] \
        + list(tf_ws) + list(tg_ws)
    return pl.pallas_call(
        body,
        grid=(n // br,),
        in_specs=[pl.BlockSpec((br, dh), lambda r: (r, 0)),
                  pl.BlockSpec((br, dh), lambda r: (r, 0)),
                  pl.BlockSpec((br, 128), lambda r: (r, 0))]
        + [cspec(a) for a in consts],
        out_specs=pl.BlockSpec((br, 2 * outdim), lambda r: (r, 0)),
        out_shape=jax.ShapeDtypeStruct((n, 2 * outdim), _F32),
    )(att4, h3, tda_pad, *consts)


def _decode_sc(emb, idx0, idx1):
    nb = idx0.shape[0]
    d2 = emb.shape[1]
    d = d2 // 2
    nchunk = d // 16
    info = plsc.get_sparse_core_info()
    nw = info.num_cores * info.num_subcores
    bpw = nb // nw
    rr = 128
    rounds = bpw // rr
    mesh = plsc.VectorSubcoreMesh(core_axis_name="c", subcore_axis_name="s")

    @functools.partial(
        pl.kernel,
        mesh=mesh,
        out_type=jax.ShapeDtypeStruct((nb,), _F32),
        scratch_types=[
            pltpu.VMEM((rr,), jnp.int32),
            pltpu.VMEM((rr,), jnp.int32),
            pltpu.VMEM((rr,), jnp.int32),
            pltpu.VMEM((rr,), jnp.int32),
            pltpu.VMEM((rr, d2), _F32),
            pltpu.VMEM((rr, d2), _F32),
            pltpu.VMEM((rr, d2), _F32),
            pltpu.VMEM((rr, d2), _F32),
            pltpu.VMEM((bpw,), _F32),
            pltpu.SemaphoreType.DMA,
            pltpu.SemaphoreType.DMA,
        ],
    )
    def k(emb_hbm, i0_hbm, i1_hbm, out_hbm, i0a, i1a, i0b, i1b,
          r0a, r1a, r0b, r1b, o_v, sema, semb):
        wid = lax.axis_index("s") * info.num_cores + lax.axis_index("c")
        base = wid * bpw
        lane = lax.broadcasted_iota(jnp.int32, (16,), 0)
        gdn = lax.GatherDimensionNumbers(
            offset_dims=(), collapsed_slice_dims=(0,), start_index_map=(0,))
        bufs = ((i0a, i1a, r0a, r1a, sema), (i0b, i1b, r0b, r1b, semb))

        def hsum16(v):
            for sh in (1, 2, 4, 8):
                perm = jnp.bitwise_xor(lane, sh)
                v = v + lax.gather(
                    v, perm[:, None], gdn, slice_sizes=(1,),
                    mode=lax.GatherScatterMode.PROMISE_IN_BOUNDS)
            return v

        def start(rnd, bset):
            i0_v, i1_v, r0_v, r1_v, sem = bset
            b0 = base + rnd * rr
            pltpu.sync_copy(i0_hbm.at[pl.ds(b0, rr)], i0_v)
            pltpu.sync_copy(i1_hbm.at[pl.ds(b0, rr)], i1_v)
            cp0 = pltpu.async_copy(emb_hbm.at[i0_v], r0_v, sem)
            cp1 = pltpu.async_copy(emb_hbm.at[i1_v], r1_v, sem)
            return cp0, cp1

        pending = start(0, bufs[0])
        for rnd in range(rounds):
            cp0, cp1 = pending
            cp0.wait()
            cp1.wait()
            if rnd + 1 < rounds:
                pending = start(rnd + 1, bufs[(rnd + 1) % 2])
            _, _, r0_v, r1_v, _ = bufs[rnd % 2]

            def group(gi, carry):
                vec = jnp.zeros((16,), _F32)
                for q in range(16):
                    p = gi * 16 + q
                    acc = r0_v[p, pl.ds(0, 16)] * r1_v[p, pl.ds(d, 16)]
                    for ch in range(1, nchunk):
                        acc = acc + (r0_v[p, pl.ds(ch * 16, 16)]
                                     * r1_v[p, pl.ds(d + ch * 16, 16)])
                    vec = jnp.where(lane == q, hsum16(acc), vec)
                o_v[pl.ds(rnd * rr + gi * 16, 16)] = vec
                return carry

            lax.fori_loop(0, rr // 16, group, 0)
        pltpu.sync_copy(o_v, out_hbm.at[pl.ds(base, bpw)])

    return k(emb, idx0, idx1)


def kernel(x, adj, train_sample, tda_feat, c1_W, c1_a, c1_b, c2_W, c2_a,
           c2_b, c3_W, c3_a, c3_b, c4_W, c4_a, c4_b, proj2_W, proj2_b,
           gate_W, gate_b, gatex_W, gatex_b, gatead_W, gatead_b, fp1_W,
           fp1_b, fp2_W, fp2_b, tf1_W, tf1_b, tf2_W, tf2_b, tf3_W, tf3_b,
           tff_W, tff_b, tf_ln1_g, tf_ln1_b, tf_ln2_g, tf_ln2_b, tf_ln3_g,
           tf_ln3_b, tg1_W, tg1_b, tg2_W, tg2_b, tg3_W, tg3_b, tgf_W,
           tgf_b, tg_ln1_g, tg_ln1_b, tg_ln2_g, tg_ln2_b, tg_ln3_g,
           tg_ln3_b):
    n = x.shape[0]

    mask8 = _mask8(adj)

    def attn(hs, b, dout):
        h0, h1, s0, s1, t0, t1 = hs
        return _attn(mask8, h0, h1, s0, s1, t0, t1, b.reshape(1, -1), dout)

    hs = _proj(None, x, c1_W, c1_a)
    att1 = attn(hs, c1_b, c1_W.shape[2])
    *hs, r2 = _proj(att1, x, c2_W, c2_a, proj2_W, proj2_b)
    att2 = attn(hs, c2_b, c2_W.shape[2])
    h2, *hs = _proj(att2, r2, c3_W, c3_a)
    att3 = attn(hs, c3_b, c3_W.shape[2])
    h3, *hs = _proj(att3, h2, c4_W, c4_a)
    att4 = attn(hs, c4_b, c4_W.shape[2])

    dh = att4.shape[1]
    dct = dh + 128
    tda_pad = _pad2(tda_feat, n, 128)
    gw = _pad2(gate_W, dct, dct)
    gb = _pad2(gate_b.reshape(1, -1), 1, dct)
    gxw = _pad2(gatex_W, dct, dct)
    gxb = _pad2(gatex_b.reshape(1, -1), 1, dct)
    gaw = _pad2(gatead_W, dct, gatead_W.shape[1])
    tf_ws = (tf1_W, tf1_b.reshape(1, -1), tf2_W, tf2_b.reshape(1, -1),
             tf3_W, tf3_b.reshape(1, -1), tff_W, tff_b.reshape(1, -1),
             tf_ln1_g.reshape(1, -1), tf_ln1_b.reshape(1, -1),
             tf_ln2_g.reshape(1, -1), tf_ln2_b.reshape(1, -1),
             tf_ln3_g.reshape(1, -1), tf_ln3_b.reshape(1, -1))
    tg_ws = (tg1_W, tg1_b.reshape(1, -1), tg2_W, tg2_b.reshape(1, -1),
             tg3_W, tg3_b.reshape(1, -1), tgf_W, tgf_b.reshape(1, -1),
             tg_ln1_g.reshape(1, -1), tg_ln1_b.reshape(1, -1),
             tg_ln2_g.reshape(1, -1), tg_ln2_b.reshape(1, -1),
             tg_ln3_g.reshape(1, -1), tg_ln3_b.reshape(1, -1))
    emb = _tail(att4, h3, tda_pad, gw, gb, gxw, gxb, gaw,
                gatead_b.reshape(1, -1), fp1_W, fp1_b.reshape(1, -1),
                fp2_W, fp2_b.reshape(1, -1), tf_ws, tg_ws)

    idx0 = train_sample[:, 0]
    idx1 = train_sample[:, 1]
    return _decode_sc(emb, idx0, idx1)


# speedup vs baseline: 1.1259x; 1.0004x over previous
import functools

import jax
import jax.numpy as jnp
from jax import lax
from jax.experimental import pallas as pl
from jax.experimental.pallas import tpu as pltpu
from jax.experimental.pallas import tpu_sc as plsc

_ALPHA = 0.2
_F32 = jnp.float32


def _leaky(x):
    return jnp.where(x >= 0, x, _ALPHA * x)


def _pad2(w, rows, cols):
    out = jnp.zeros((rows, cols), w.dtype)
    return out.at[: w.shape[0], : w.shape[1]].set(w)


def _mask8(adj):
    n = adj.shape[0]
    br = 512 if n % 512 == 0 else n

    def body(a_ref, m_ref):
        m_ref[...] = (a_ref[...] != 0.0).astype(jnp.int8)

    return pl.pallas_call(
        body,
        grid=(n // br,),
        in_specs=[pl.BlockSpec((br, n), lambda r: (r, 0))],
        out_specs=pl.BlockSpec((br, n), lambda r: (r, 0)),
        out_shape=jax.ShapeDtypeStruct((n, n), jnp.int8),
    )(adj)


def _proj(att, res, W, a, p2w=None, p2b=None):
    n, din = res.shape
    _, _, dout = W.shape
    br = 512 if n % 512 == 0 else n
    bf = jnp.bfloat16
    a1 = a[:, :dout, 0]
    a2 = a[:, dout:, 0]
    has_att = att is not None
    has_p2 = p2w is not None
    emit_comb = has_att and not has_p2

    def body(*refs):
        i = 0
        if has_att:
            att_ref = refs[i]
            i += 1
        res_ref = refs[i]
        w0, w1, a10, a11, a20, a21 = refs[i + 1:i + 7]
        i += 7
        if has_p2:
            p2w_ref, p2b_ref = refs[i:i + 2]
            i += 2
        if emit_comb:
            comb_ref = refs[i]
            i += 1
        h0, h1, s0, s1, t0, t1 = refs[i:i + 6]
        i += 6
        if has_p2:
            r2_ref = refs[i]

        if has_att:
            av = att_ref[...]
            xv = jnp.where(av > 0, av, jnp.exp(av) - 1.0) + res_ref[...]
            if emit_comb:
                comb_ref[...] = xv
        else:
            xv = res_ref[...]
        xb = xv.astype(bf)
        for w_ref, a1_ref, a2_ref, h_ref, s_ref, t_ref in (
            (w0, a10, a20, h0, s0, t0),
            (w1, a11, a21, h1, s1, t1),
        ):
            hk = jnp.dot(xb, w_ref[...], preferred_element_type=_F32)
            h_ref[...] = hk.astype(bf)
            s_ref[0, :] = jnp.sum(hk * a1_ref[...], axis=1)
            t_ref[0, :] = jnp.sum(hk * a2_ref[...], axis=1)
        if has_p2:
            r2_ref[...] = (jnp.dot(xv, p2w_ref[...],
                                   preferred_element_type=_F32)
                           + p2b_ref[...])

    xspec = pl.BlockSpec((br, din), lambda r: (r, 0))
    wspec = pl.BlockSpec((din, dout), lambda r: (0, 0))
    aspec = pl.BlockSpec((1, dout), lambda r: (0, 0))
    hspec = pl.BlockSpec((br, dout), lambda r: (r, 0))
    vspec = pl.BlockSpec((1, br), lambda r: (0, r))

    in_specs = ([xspec] if has_att else []) + [
        xspec, wspec, wspec, aspec, aspec, aspec, aspec]
    args = ([att] if has_att else []) + [
        res, W.astype(bf)[0], W.astype(bf)[1],
        a1[0:1], a1[1:2], a2[0:1], a2[1:2]]
    out_specs = [hspec, hspec, vspec, vspec, vspec, vspec]
    out_shape = [jax.ShapeDtypeStruct((n, dout), bf)] * 2 \
        + [jax.ShapeDtypeStruct((1, n), _F32)] * 4
    if emit_comb:
        out_specs = [xspec] + out_specs
        out_shape = [jax.ShapeDtypeStruct((n, din), _F32)] + out_shape
    if has_p2:
        dp = p2w.shape[1]
        in_specs += [pl.BlockSpec((din, dp), lambda r: (0, 0)),
                     pl.BlockSpec((1, dp), lambda r: (0, 0))]
        args += [p2w, p2b.reshape(1, -1)]
        out_specs = out_specs + [pl.BlockSpec((br, dp), lambda r: (r, 0))]
        out_shape = out_shape + [jax.ShapeDtypeStruct((n, dp), _F32)]

    return pl.pallas_call(
        body,
        grid=(n // br,),
        in_specs=in_specs,
        out_specs=out_specs,
        out_shape=out_shape,
    )(*args)


def _attn(adj, h0, h1, s0, s1, t0, t1, bcat, dout):
    n = adj.shape[0]
    br = 2048 if n % 2048 == 0 else n
    bc = 2048 if n % 2048 == 0 else n
    grid = (n // br, n // bc)
    twod = 2 * dout

    def body(adj_ref, h0_ref, h1_ref, s0_ref, s1_ref, t0_ref, t1_ref, b_ref,
             out_ref, acc0, acc1, den0, den1):
        r = pl.program_id(0)
        c = pl.program_id(1)
        ncols = pl.num_programs(1)

        @pl.when(c == 0)
        def _():
            acc0[...] = jnp.zeros_like(acc0)
            acc1[...] = jnp.zeros_like(acc1)
            den0[...] = jnp.zeros_like(den0)
            den1[...] = jnp.zeros_like(den1)

        mask = adj_ref[...] != 0
        for s_ref, t_ref, h_ref, acc, den in (
            (s0_ref, t0_ref, h0_ref, acc0, den0),
            (s1_ref, t1_ref, h1_ref, acc1, den1),
        ):
            srow = s_ref[0, pl.ds(r * br, br)]
            tcol = t_ref[0, pl.ds(c * bc, bc)]
            tmax = jnp.max(t_ref[0, :])
            m = jnp.maximum(_leaky(srow + tmax), 0.0)
            bf = jnp.bfloat16
            e1 = jnp.exp(srow + tmax - m).astype(bf)
            e2 = jnp.exp(_ALPHA * (srow + tmax) - m).astype(bf)
            em = jnp.exp(-m).astype(bf)
            f1 = jnp.exp(tcol - tmax).astype(bf)
            f2 = jnp.exp(_ALPHA * (tcol - tmax)).astype(bf)
            p = jnp.maximum(e1[:, None] * f1[None, :],
                            e2[:, None] * f2[None, :])
            p = jnp.where(mask, p, em[:, None])
            acc[...] += jnp.dot(p, h_ref[...], preferred_element_type=_F32)
            den[...] += jnp.sum(p, axis=1, keepdims=True, dtype=_F32)

        @pl.when(c == ncols - 1)
        def _():
            for k, (acc, den) in enumerate(((acc0, den0), (acc1, den1))):
                o = _leaky(acc[...] / den[:, :1])
                nrm = jnp.maximum(
                    jnp.sqrt(jnp.sum(o * o, axis=1, keepdims=True)), 1e-12)
                out_ref[:, k * dout:(k + 1) * dout] = (
                    o / nrm + b_ref[0, k * dout:(k + 1) * dout][None, :])

    hspec = pl.BlockSpec((bc, dout), lambda r, c: (c, 0))
    vspec = pl.BlockSpec((1, n), lambda r, c: (0, 0))
    return pl.pallas_call(
        body,
        grid=grid,
        in_specs=[pl.BlockSpec((br, bc), lambda r, c: (r, c)),
                  hspec, hspec, vspec, vspec, vspec, vspec,
                  pl.BlockSpec((1, twod), lambda r, c: (0, 0))],
        out_specs=pl.BlockSpec((br, twod), lambda r, c: (r, 0)),
        out_shape=jax.ShapeDtypeStruct((n, twod), _F32),
        scratch_shapes=[
            pltpu.VMEM((br, dout), _F32),
            pltpu.VMEM((br, dout), _F32),
            pltpu.VMEM((br, 128), _F32),
            pltpu.VMEM((br, 128), _F32),
        ],
        compiler_params=pltpu.CompilerParams(
            dimension_semantics=("arbitrary", "arbitrary")),
    )(adj, h0, h1, s0, s1, t0, t1, bcat)


def _tail(att4, h3, tda_pad, gw, gb, gxw, gxb, gaw, gab, f1w, f1b, f2w, f2b,
          tf_ws, tg_ws):
    n, dh = att4.shape
    outdim = tf_ws[6].shape[1]
    br = 512 if n % 512 == 0 else n
    bf = jnp.bfloat16

    def bdot(x, w_ref):
        return jnp.dot(x, w_ref[...], preferred_element_type=_F32)

    def body(a4_ref, h3_ref, td_ref, gw_ref, gb_ref, gxw_ref, gxb_ref,
             gaw_ref, gab_ref, f1w_ref, f1b_ref, f2w_ref, f2b_ref,
             aw1, ab1, aw2, ab2, aw3, ab3, awf, abf, ag1, an1, ag2, an2,
             ag3, an3,
             bw1, bb1, bw2, bb2, bw3, bb3, bwf, bbf, bg1, bn1, bg2, bn2,
             bg3, bn3,
             emb_ref):
        av = a4_ref[...]
        h4 = jnp.where(av > 0, av, jnp.exp(av) - 1.0) + h3_ref[...]
        ct = jnp.concatenate([h4, td_ref[...]], axis=1)
        g = jax.nn.sigmoid(bdot(ct, gw_ref) + gb_ref[...])
        v = bdot(ct, gxw_ref) + gxb_ref[...]
        fused = bdot(g * v, gaw_ref) + gab_ref[...]
        mid = _leaky(bdot(fused, f1w_ref) + f1b_ref[...])
        fused = fused + bdot(mid, f2w_ref) + f2b_ref[...]

        def ln(xx, g_ref, b_ref):
            mu = jnp.mean(xx, axis=1, keepdims=True)
            xc = xx - mu
            var = jnp.mean(xc * xc, axis=1, keepdims=True)
            return xc / jnp.sqrt(var + 1e-5) * g_ref[...] + b_ref[...]

        def branch(w1, b1, w2, b2, w3, b3, wf, bfi, g1, n1, g2, n2, g3, n3,
                   off):
            t = _leaky(ln(bdot(fused, w1) + b1[...], g1, n1))
            t = _leaky(ln(bdot(t, w2) + b2[...], g2, n2))
            t = _leaky(ln(bdot(t, w3) + b3[...], g3, n3))
            emb_ref[:, off:off + outdim] = bdot(t, wf) + bfi[...]

        branch(aw1, ab1, aw2, ab2, aw3, ab3, awf, abf, ag1, an1, ag2, an2,
               ag3, an3, 0)
        branch(bw1, bb1, bw2, bb2, bw3, bb3, bwf, bbf, bg1, bn1, bg2, bn2,
               bg3, bn3, outdim)

    def cspec(arr):
        return pl.BlockSpec(arr.shape, lambda r: tuple(0 for _ in arr.shape))

    consts = [gw, gb, gxw, gxb, gaw, gab, f1w, f1b, f2w, f2b] \
        + list(tf_ws) + list(tg_ws)
    return pl.pallas_call(
        body,
        grid=(n // br,),
        in_specs=[pl.BlockSpec((br, dh), lambda r: (r, 0)),
                  pl.BlockSpec((br, dh), lambda r: (r, 0)),
                  pl.BlockSpec((br, 128), lambda r: (r, 0))]
        + [cspec(a) for a in consts],
        out_specs=pl.BlockSpec((br, 2 * outdim), lambda r: (r, 0)),
        out_shape=jax.ShapeDtypeStruct((n, 2 * outdim), _F32),
    )(att4, h3, tda_pad, *consts)


def _decode_sc(emb, idx0, idx1):
    nb = idx0.shape[0]
    d2 = emb.shape[1]
    d = d2 // 2
    nchunk = d // 16
    info = plsc.get_sparse_core_info()
    nw = info.num_cores * info.num_subcores
    bpw = nb // nw
    rr = 128
    rounds = bpw // rr
    mesh = plsc.VectorSubcoreMesh(core_axis_name="c", subcore_axis_name="s")

    @functools.partial(
        pl.kernel,
        mesh=mesh,
        out_type=jax.ShapeDtypeStruct((nb,), _F32),
        scratch_types=[
            pltpu.VMEM((rr,), jnp.int32),
            pltpu.VMEM((rr,), jnp.int32),
            pltpu.VMEM((rr,), jnp.int32),
            pltpu.VMEM((rr,), jnp.int32),
            pltpu.VMEM((rr, d2), _F32),
            pltpu.VMEM((rr, d2), _F32),
            pltpu.VMEM((rr, d2), _F32),
            pltpu.VMEM((rr, d2), _F32),
            pltpu.VMEM((bpw,), _F32),
            pltpu.SemaphoreType.DMA,
            pltpu.SemaphoreType.DMA,
        ],
    )
    def k(emb_hbm, i0_hbm, i1_hbm, out_hbm, i0a, i1a, i0b, i1b,
          r0a, r1a, r0b, r1b, o_v, sema, semb):
        wid = lax.axis_index("s") * info.num_cores + lax.axis_index("c")
        base = wid * bpw
        lane = lax.broadcasted_iota(jnp.int32, (16,), 0)
        gdn = lax.GatherDimensionNumbers(
            offset_dims=(), collapsed_slice_dims=(0,), start_index_map=(0,))
        bufs = ((i0a, i1a, r0a, r1a, sema), (i0b, i1b, r0b, r1b, semb))

        def hsum16(v):
            for sh in (1, 2, 4, 8):
                perm = jnp.bitwise_xor(lane, sh)
                v = v + lax.gather(
                    v, perm[:, None], gdn, slice_sizes=(1,),
                    mode=lax.GatherScatterMode.PROMISE_IN_BOUNDS)
            return v

        def start(rnd, bset):
            i0_v, i1_v, r0_v, r1_v, sem = bset
            b0 = base + rnd * rr
            pltpu.sync_copy(i0_hbm.at[pl.ds(b0, rr)], i0_v)
            pltpu.sync_copy(i1_hbm.at[pl.ds(b0, rr)], i1_v)
            cp0 = pltpu.async_copy(emb_hbm.at[i0_v], r0_v, sem)
            cp1 = pltpu.async_copy(emb_hbm.at[i1_v], r1_v, sem)
            return cp0, cp1

        pending = start(0, bufs[0])
        for rnd in range(rounds):
            cp0, cp1 = pending
            cp0.wait()
            cp1.wait()
            if rnd + 1 < rounds:
                pending = start(rnd + 1, bufs[(rnd + 1) % 2])
            _, _, r0_v, r1_v, _ = bufs[rnd % 2]

            def group(gi, carry):
                vec = jnp.zeros((16,), _F32)
                for q in range(16):
                    p = gi * 16 + q
                    acc = r0_v[p, pl.ds(0, 16)] * r1_v[p, pl.ds(d, 16)]
                    for ch in range(1, nchunk):
                        acc = acc + (r0_v[p, pl.ds(ch * 16, 16)]
                                     * r1_v[p, pl.ds(d + ch * 16, 16)])
                    vec = jnp.where(lane == q, hsum16(acc), vec)
                o_v[pl.ds(rnd * rr + gi * 16, 16)] = vec
                return carry

            lax.fori_loop(0, rr // 16, group, 0)
        pltpu.sync_copy(o_v, out_hbm.at[pl.ds(base, bpw)])

    return k(emb, idx0, idx1)


def kernel(x, adj, train_sample, tda_feat, c1_W, c1_a, c1_b, c2_W, c2_a,
           c2_b, c3_W, c3_a, c3_b, c4_W, c4_a, c4_b, proj2_W, proj2_b,
           gate_W, gate_b, gatex_W, gatex_b, gatead_W, gatead_b, fp1_W,
           fp1_b, fp2_W, fp2_b, tf1_W, tf1_b, tf2_W, tf2_b, tf3_W, tf3_b,
           tff_W, tff_b, tf_ln1_g, tf_ln1_b, tf_ln2_g, tf_ln2_b, tf_ln3_g,
           tf_ln3_b, tg1_W, tg1_b, tg2_W, tg2_b, tg3_W, tg3_b, tgf_W,
           tgf_b, tg_ln1_g, tg_ln1_b, tg_ln2_g, tg_ln2_b, tg_ln3_g,
           tg_ln3_b):
    n = x.shape[0]

    mask8 = _mask8(adj)

    def attn(hs, b, dout):
        h0, h1, s0, s1, t0, t1 = hs
        return _attn(mask8, h0, h1, s0, s1, t0, t1, b.reshape(1, -1), dout)

    hs = _proj(None, x, c1_W, c1_a)
    att1 = attn(hs, c1_b, c1_W.shape[2])
    *hs, r2 = _proj(att1, x, c2_W, c2_a, proj2_W, proj2_b)
    att2 = attn(hs, c2_b, c2_W.shape[2])
    h2, *hs = _proj(att2, r2, c3_W, c3_a)
    att3 = attn(hs, c3_b, c3_W.shape[2])
    h3, *hs = _proj(att3, h2, c4_W, c4_a)
    att4 = attn(hs, c4_b, c4_W.shape[2])

    dh = att4.shape[1]
    dct = dh + 128
    tda_pad = _pad2(tda_feat, n, 128)
    gw = _pad2(gate_W, dct, dct)
    gb = _pad2(gate_b.reshape(1, -1), 1, dct)
    gxw = _pad2(gatex_W, dct, dct)
    gxb = _pad2(gatex_b.reshape(1, -1), 1, dct)
    gaw = _pad2(gatead_W, dct, gatead_W.shape[1])
    tf_ws = (tf1_W, tf1_b.reshape(1, -1), tf2_W, tf2_b.reshape(1, -1),
             tf3_W, tf3_b.reshape(1, -1), tff_W, tff_b.reshape(1, -1),
             tf_ln1_g.reshape(1, -1), tf_ln1_b.reshape(1, -1),
             tf_ln2_g.reshape(1, -1), tf_ln2_b.reshape(1, -1),
             tf_ln3_g.reshape(1, -1), tf_ln3_b.reshape(1, -1))
    tg_ws = (tg1_W, tg1_b.reshape(1, -1), tg2_W, tg2_b.reshape(1, -1),
             tg3_W, tg3_b.reshape(1, -1), tgf_W, tgf_b.reshape(1, -1),
             tg_ln1_g.reshape(1, -1), tg_ln1_b.reshape(1, -1),
             tg_ln2_g.reshape(1, -1), tg_ln2_b.reshape(1, -1),
             tg_ln3_g.reshape(1, -1), tg_ln3_b.reshape(1, -1))
    emb = _tail(att4, h3, tda_pad, gw, gb, gxw, gxb, gaw,
                gatead_b.reshape(1, -1), fp1_W, fp1_b.reshape(1, -1),
                fp2_W, fp2_b.reshape(1, -1), tf_ws, tg_ws)

    idx0 = train_sample[:, 0]
    idx1 = train_sample[:, 1]
    return _decode_sc(emb, idx0, idx1)
